# Initial kernel scaffold; baseline (speedup 1.0000x reference)
import jax
import jax.numpy as jnp
from jax.experimental import pallas as pl


def kernel(x, edge_index, batch, W1, b1, W2, b2, Wl1, bl1, Wl2, bl2):
    raise NotImplementedError("write your pallas kernel here")


# speedup vs baseline: 11.1221x; 11.1221x over previous
import functools

import jax
import jax.numpy as jnp
from jax import lax
from jax.experimental import pallas as pl
from jax.experimental.pallas import tpu as pltpu, tpu_sc as plsc

N = 50000
E = 800000
G = 32

NP = 50176
Bn = 1568
NB = NP // Bn
Ep = 819200
R = Ep // 128
RW = R // 32
KB = 8
NBLK = RW // KB
APT = NP // 16

@functools.cache
def _mesh():
    return plsc.VectorSubcoreMesh(core_axis_name="c", subcore_axis_name="s",
                                  num_cores=2, num_subcores=16)


@functools.cache
def _sc_deg():
    @functools.partial(
        pl.kernel,
        out_type=jax.ShapeDtypeStruct((2, NP, 16), jnp.float32),
        mesh=_mesh(),
        compiler_params=pltpu.CompilerParams(use_tc_tiling_on_sc=False),
        scratch_types=[
            pltpu.VMEM((KB, 128), jnp.int32),
            pltpu.VMEM((KB, 128, 16), jnp.float32),
            pltpu.VMEM_SHARED((NP, 16), jnp.float32),
            pltpu.SemaphoreType.DMA,
        ],
    )
    def sc_deg(dst_h, ones_h, zeros_h, out_h, idx_d, ones_v, acc, sem):
        cid = lax.axis_index("c")
        sid = lax.axis_index("s")
        wrow = (cid * 16 + sid) * RW
        arow = sid * APT
        pltpu.sync_copy(zeros_h.at[pl.ds(arow, APT)], acc.at[pl.ds(arow, APT)])
        pltpu.sync_copy(ones_h, ones_v)
        plsc.subcore_barrier()

        def blk(i, carry):
            r0 = pl.multiple_of(wrow + i * KB, 8)
            pltpu.sync_copy(dst_h.at[pl.ds(r0, KB)], idx_d)
            descs = [pltpu.async_copy(ones_v.at[j], acc.at[idx_d.at[j]],
                                      sem, add=True) for j in range(KB)]
            for d in descs:
                d.wait()
            return carry

        lax.fori_loop(0, NBLK, blk, 0)
        plsc.subcore_barrier()
        pltpu.sync_copy(acc.at[pl.ds(arow, APT)],
                        out_h.at[cid].at[pl.ds(arow, APT)])

    return sc_deg


@functools.cache
def _sc_agg(nchunks):
    @functools.partial(
        pl.kernel,
        out_type=jax.ShapeDtypeStruct((2, nchunks, NP, 16), jnp.float32),
        mesh=_mesh(),
        compiler_params=pltpu.CompilerParams(use_tc_tiling_on_sc=False),
        scratch_types=[
            pltpu.VMEM((KB, 128), jnp.int32),
            pltpu.VMEM((KB, 128), jnp.int32),
            pltpu.VMEM((KB, 128, 16), jnp.float32),
            pltpu.VMEM_SHARED((NP, 16), jnp.float32),
            pltpu.SemaphoreType.DMA,
        ],
    )
    def sc_agg(*refs):
        tables = refs[:nchunks]
        src_h, dst_h, zeros_h, out_h, idx_s, idx_d, rows, acc, sem = refs[nchunks:]
        cid = lax.axis_index("c")
        sid = lax.axis_index("s")
        wrow = (cid * 16 + sid) * RW
        arow = sid * APT
        for c in range(nchunks):
            pltpu.sync_copy(zeros_h.at[pl.ds(arow, APT)],
                            acc.at[pl.ds(arow, APT)])
            plsc.subcore_barrier()

            def blk(i, carry, table=tables[c]):
                r0 = pl.multiple_of(wrow + i * KB, 8)
                pltpu.sync_copy(src_h.at[pl.ds(r0, KB)], idx_s)
                pltpu.sync_copy(dst_h.at[pl.ds(r0, KB)], idx_d)
                gd = [pltpu.async_copy(table.at[idx_s.at[j]], rows.at[j],
                                       sem) for j in range(KB)]
                for d in gd:
                    d.wait()
                sd = [pltpu.async_copy(rows.at[j], acc.at[idx_d.at[j]],
                                       sem, add=True) for j in range(KB)]
                for d in sd:
                    d.wait()
                return carry

            lax.fori_loop(0, NBLK, blk, 0)
            plsc.subcore_barrier()
            pltpu.sync_copy(acc.at[pl.ds(arow, APT)],
                            out_h.at[cid].at[c].at[pl.ds(arow, APT)])
            plsc.subcore_barrier()

    return sc_agg


def _dis_of(dp):
    deg = dp[0, :, 0:1] + dp[1, :, 0:1] + 1.0
    return lax.rsqrt(deg)


def _tc_prep_body(dp_ref, xp_ref, xs_ref):
    dis = _dis_of(dp_ref[...])
    xs_ref[...] = xp_ref[...] * dis


@functools.cache
def _tc_kernels(interpret=False):
    tc_prep = pl.pallas_call(
        _tc_prep_body,
        interpret=interpret,
        grid=(NB,),
    in_specs=[
        pl.BlockSpec((2, Bn, 16), lambda i: (0, i, 0)),
        pl.BlockSpec((Bn, 16), lambda i: (i, 0)),
    ],
    out_specs=pl.BlockSpec((Bn, 16), lambda i: (i, 0)),
    out_shape=jax.ShapeDtypeStruct((NP, 16), jnp.float32),
    )
    tc_mid = pl.pallas_call(
        _tc_mid_body,
        interpret=interpret,
        grid=(NB,),
        in_specs=[
            pl.BlockSpec((Bn, 16), lambda i: (i, 0)),
            pl.BlockSpec((2, 1, Bn, 16), lambda i: (0, 0, i, 0)),
            pl.BlockSpec((2, Bn, 16), lambda i: (0, i, 0)),
            pl.BlockSpec((16, 112), lambda i: (0, 0)),
            pl.BlockSpec((1, 112), lambda i: (0, 0)),
        ],
        out_specs=[pl.BlockSpec((Bn, 16), lambda i: (i, 0)) for _ in range(7)],
        out_shape=[jax.ShapeDtypeStruct((NP, 16), jnp.float32) for _ in range(7)],
    )
    tc_final = pl.pallas_call(
        _tc_final_body,
        interpret=interpret,
        grid=(NB,),
        in_specs=(
            [pl.BlockSpec((Bn, 16), lambda i: (i, 0)) for _ in range(7)] + [
                pl.BlockSpec((2, 7, Bn, 16), lambda i: (0, 0, i, 0)),
                pl.BlockSpec((2, Bn, 16), lambda i: (0, i, 0)),
                pl.BlockSpec((1, 1, Bn), lambda i: (i, 0, 0)),
                pl.BlockSpec((112, 200), lambda i: (0, 0)),
                pl.BlockSpec((1, 200), lambda i: (0, 0)),
                pl.BlockSpec((200, 100), lambda i: (0, 0)),
                pl.BlockSpec((1, 100), lambda i: (0, 0)),
                pl.BlockSpec((100, 128), lambda i: (0, 0)),
                pl.BlockSpec((1, 128), lambda i: (0, 0)),
            ]
        ),
        out_specs=pl.BlockSpec((G, 128), lambda i: (0, 0)),
        out_shape=jax.ShapeDtypeStruct((G, 128), jnp.float32),
        scratch_shapes=[
            pltpu.VMEM((G, 200), jnp.float32),
            pltpu.VMEM((G, 128), jnp.float32),
        ],
    )
    return tc_prep, tc_mid, tc_final


def _tc_mid_body(xs_ref, ax_ref, dp_ref, w_ref, b_ref, *out_refs):
    dis = _dis_of(dp_ref[...])
    a1 = (xs_ref[...] + ax_ref[0, 0] + ax_ref[1, 0]) * dis
    h = jnp.dot(a1, w_ref[...], preferred_element_type=jnp.float32) + b_ref[...]
    h = h * jax.nn.sigmoid(h)
    hs = h * dis
    for c in range(7):
        out_refs[c][...] = hs[:, 16 * c:16 * (c + 1)]


def _tc_final_body(t0, t1, t2, t3, t4, t5, t6, ah_ref, dp_ref, bt_ref,
                   w2_ref, b2_ref, wl1_ref, bl1_ref, wl2_ref, bl2_ref,
                   out_ref, sums, cnt):
    i = pl.program_id(0)

    @pl.when(i == 0)
    def _():
        sums[...] = jnp.zeros_like(sums)
        cnt[...] = jnp.zeros_like(cnt)

    dis = _dis_of(dp_ref[...])
    ts = (t0, t1, t2, t3, t4, t5, t6)
    parts = [ts[c][...] + ah_ref[0, c] + ah_ref[1, c] for c in range(7)]
    a2 = jnp.concatenate(parts, axis=1) * dis
    h2 = jnp.dot(a2, w2_ref[...], preferred_element_type=jnp.float32) + b2_ref[...]
    h2 = h2 * jax.nn.sigmoid(h2)
    b = bt_ref[0, 0, :]
    gids = lax.broadcasted_iota(jnp.int32, (G, Bn), 0)
    oh = (gids == b[None, :]).astype(jnp.float32)
    sums[...] += jnp.dot(oh, h2, preferred_element_type=jnp.float32)
    cnt[...] += jnp.broadcast_to(
        jnp.sum(oh, axis=1, keepdims=True), (G, 128))

    @pl.when(i == NB - 1)
    def _():
        pooled = sums[...] / jnp.maximum(cnt[:, 0:1], 1.0)
        z = jnp.dot(pooled, wl1_ref[...],
                    preferred_element_type=jnp.float32) + bl1_ref[...]
        z = z * jax.nn.sigmoid(z)
        o = jnp.dot(z, wl2_ref[...],
                    preferred_element_type=jnp.float32) + bl2_ref[...]
        out_ref[...] = o


def kernel(x, edge_index, batch, W1, b1, W2, b2, Wl1, bl1, Wl2, bl2):
    f32 = jnp.float32
    src = edge_index[0]
    dst = edge_index[1]
    epad = jnp.full((Ep - E,), N, jnp.int32)
    srcp = jnp.concatenate([src, epad]).reshape(R, 128)
    dstp = jnp.concatenate([dst, epad]).reshape(R, 128)
    xpad = jnp.zeros((NP, 16), f32).at[:N, :3].set(x.astype(f32))
    zeros16 = jnp.zeros((NP, 16), f32)
    onesv = jnp.ones((KB, 128, 16), f32)
    batchp = jnp.concatenate(
        [batch, jnp.full((NP - N,), G, jnp.int32)]).reshape(NB, 1, Bn)
    W1p = jnp.zeros((16, 112), f32).at[:3, :100].set(W1)
    b1p = jnp.zeros((1, 112), f32).at[0, :100].set(b1)
    W2p = jnp.zeros((112, 200), f32).at[:100, :].set(W2)
    b2r = b2.reshape(1, 200)
    bl1r = bl1.reshape(1, 100)
    Wl2p = jnp.zeros((100, 128), f32).at[:, 0:1].set(Wl2)
    bl2p = jnp.broadcast_to(bl2.reshape(1, 1), (1, 128))

    tc_prep, tc_mid, tc_final = _tc_kernels()
    degpart = _sc_deg()(dstp, onesv, zeros16)
    xs = tc_prep(degpart, xpad)
    aggx = _sc_agg(1)(xs, srcp, dstp, zeros16)
    ts = tc_mid(xs, aggx, degpart, W1p, b1p)
    aggh = _sc_agg(7)(*ts, srcp, dstp, zeros16)
    out = tc_final(*ts, aggh, degpart, batchp, W2p, b2r, Wl1, bl1r,
                   Wl2p, bl2p)
    return out[:, 0]


# speedup vs baseline: 13.6710x; 1.2292x over previous
import functools

import jax
import jax.numpy as jnp
from jax import lax
from jax.experimental import pallas as pl
from jax.experimental.pallas import tpu as pltpu, tpu_sc as plsc

N = 50000
E = 800000
G = 32

NP = 50176
Bn = 1568
NB = NP // Bn
Ep = 819200
R = Ep // 128
RW = R // 32
KB = 4
NIT = RW // (2 * KB)
APT = NP // 16


@functools.cache
def _mesh():
    return plsc.VectorSubcoreMesh(core_axis_name="c", subcore_axis_name="s",
                                  num_cores=2, num_subcores=16)


@functools.cache
def _sc_pass(nchunks, w, gather):
    scratch = [pltpu.VMEM((RW, 128), jnp.int32)]
    if gather:
        scratch.append(pltpu.VMEM((RW, 128), jnp.int32))
    scratch += [
        pltpu.VMEM((KB, 128, w), jnp.float32),
        pltpu.VMEM((KB, 128, w), jnp.float32),
        pltpu.VMEM_SHARED((NP, w), jnp.float32),
        pltpu.SemaphoreType.DMA,
        pltpu.SemaphoreType.DMA,
        pltpu.SemaphoreType.DMA,
        pltpu.SemaphoreType.DMA,
    ]

    @functools.partial(
        pl.kernel,
        out_type=jax.ShapeDtypeStruct((2, nchunks, NP, w), jnp.float32),
        mesh=_mesh(),
        compiler_params=pltpu.CompilerParams(use_tc_tiling_on_sc=False),
        scratch_types=scratch,
    )
    def sc_pass(*refs):
        if gather:
            tables = refs[:nchunks]
            (src_h, dst_h, zeros_h, out_h, idxd_all, idxs_all,
             rows_a, rows_b, acc, g_a, g_b, s_a, s_b) = refs[nchunks:]
        else:
            (dst_h, ones_h, zeros_h, out_h, idxd_all,
             rows_a, rows_b, acc, g_a, g_b, s_a, s_b) = refs
        cid = lax.axis_index("c")
        sid = lax.axis_index("s")
        wrow = pl.multiple_of((cid * 16 + sid) * RW, 8)
        arow = pl.multiple_of(sid * APT, 8)

        pltpu.sync_copy(dst_h.at[pl.ds(wrow, RW)], idxd_all)
        if gather:
            pltpu.sync_copy(src_h.at[pl.ds(wrow, RW)], idxs_all)
        else:
            pltpu.sync_copy(ones_h, rows_a)
            pltpu.sync_copy(ones_h, rows_b)

        def drain_s(rows, sem):
            for _ in range(KB):
                pltpu.make_async_copy(rows.at[0], acc.at[idxd_all.at[0]],
                                      sem).wait()

        for c in range(nchunks):
            pltpu.sync_copy(zeros_h.at[pl.ds(arow, APT)],
                            acc.at[pl.ds(arow, APT)])
            plsc.subcore_barrier()

            table = tables[c] if gather else None

            def it(i, carry, table=table):
                r_a = i * 2 * KB
                r_b = r_a + KB
                if gather:
                    @pl.when(i > 0)
                    def _():
                        drain_s(rows_a, s_a)
                    for j in range(KB):
                        pltpu.async_copy(table.at[idxs_all.at[r_a + j]],
                                         rows_a.at[j], g_a)

                    @pl.when(i > 0)
                    def _():
                        drain_s(rows_b, s_b)
                    for j in range(KB):
                        pltpu.async_copy(table.at[idxs_all.at[r_b + j]],
                                         rows_b.at[j], g_b)
                    for j in range(KB):
                        pltpu.make_async_copy(table.at[idxs_all.at[r_a + j]],
                                              rows_a.at[j], g_a).wait()
                    for j in range(KB):
                        pltpu.async_copy(rows_a.at[j],
                                         acc.at[idxd_all.at[r_a + j]],
                                         s_a, add=True)
                    for j in range(KB):
                        pltpu.make_async_copy(table.at[idxs_all.at[r_b + j]],
                                              rows_b.at[j], g_b).wait()
                    for j in range(KB):
                        pltpu.async_copy(rows_b.at[j],
                                         acc.at[idxd_all.at[r_b + j]],
                                         s_b, add=True)
                else:
                    @pl.when(i > 0)
                    def _():
                        drain_s(rows_a, s_a)
                        drain_s(rows_b, s_b)
                    for j in range(KB):
                        pltpu.async_copy(rows_a.at[j],
                                         acc.at[idxd_all.at[r_a + j]],
                                         s_a, add=True)
                    for j in range(KB):
                        pltpu.async_copy(rows_b.at[j],
                                         acc.at[idxd_all.at[r_b + j]],
                                         s_b, add=True)
                return carry

            lax.fori_loop(0, NIT, it, 0)
            drain_s(rows_a, s_a)
            drain_s(rows_b, s_b)
            plsc.subcore_barrier()
            pltpu.sync_copy(acc.at[pl.ds(arow, APT)],
                            out_h.at[cid].at[c].at[pl.ds(arow, APT)])
            plsc.subcore_barrier()

    return sc_pass


def _dis_of(dp):
    deg = dp[0, 0, :, 0:1] + dp[1, 0, :, 0:1] + 1.0
    return lax.rsqrt(deg)


def _tc_prep_body(dp_ref, xp_ref, xs_ref):
    dis = _dis_of(dp_ref[...])
    xs_ref[...] = xp_ref[...] * dis


def _tc_mid_body(xs_ref, ax_ref, dp_ref, w_ref, b_ref, *out_refs):
    dis = _dis_of(dp_ref[...])
    a1 = (xs_ref[...] + ax_ref[0, 0] + ax_ref[1, 0]) * dis
    h = jnp.dot(a1, w_ref[...], preferred_element_type=jnp.float32) + b_ref[...]
    h = h * jax.nn.sigmoid(h)
    hs = h * dis
    for c in range(7):
        out_refs[c][...] = hs[:, 16 * c:16 * (c + 1)]


def _tc_final_body(t0, t1, t2, t3, t4, t5, t6, ah_ref, dp_ref, bt_ref,
                   w2_ref, b2_ref, wl1_ref, bl1_ref, wl2_ref, bl2_ref,
                   out_ref, sums, cnt):
    i = pl.program_id(0)

    @pl.when(i == 0)
    def _():
        sums[...] = jnp.zeros_like(sums)
        cnt[...] = jnp.zeros_like(cnt)

    dis = _dis_of(dp_ref[...])
    ts = (t0, t1, t2, t3, t4, t5, t6)
    parts = [ts[c][...] + ah_ref[0, c] + ah_ref[1, c] for c in range(7)]
    a2 = jnp.concatenate(parts, axis=1) * dis
    h2 = jnp.dot(a2, w2_ref[...], preferred_element_type=jnp.float32) + b2_ref[...]
    h2 = h2 * jax.nn.sigmoid(h2)
    b = bt_ref[0, 0, :]
    gids = lax.broadcasted_iota(jnp.int32, (G, Bn), 0)
    oh = (gids == b[None, :]).astype(jnp.float32)
    sums[...] += jnp.dot(oh, h2, preferred_element_type=jnp.float32)
    cnt[...] += jnp.broadcast_to(
        jnp.sum(oh, axis=1, keepdims=True), (G, 128))

    @pl.when(i == NB - 1)
    def _():
        pooled = sums[...] / jnp.maximum(cnt[:, 0:1], 1.0)
        z = jnp.dot(pooled, wl1_ref[...],
                    preferred_element_type=jnp.float32) + bl1_ref[...]
        z = z * jax.nn.sigmoid(z)
        o = jnp.dot(z, wl2_ref[...],
                    preferred_element_type=jnp.float32) + bl2_ref[...]
        out_ref[...] = o


@functools.cache
def _tc_kernels(interpret=False):
    tc_prep = pl.pallas_call(
        _tc_prep_body,
        interpret=interpret,
        grid=(NB,),
        in_specs=[
            pl.BlockSpec((2, 1, Bn, 16), lambda i: (0, 0, i, 0)),
            pl.BlockSpec((Bn, 16), lambda i: (i, 0)),
        ],
        out_specs=pl.BlockSpec((Bn, 16), lambda i: (i, 0)),
        out_shape=jax.ShapeDtypeStruct((NP, 16), jnp.float32),
    )
    tc_mid = pl.pallas_call(
        _tc_mid_body,
        interpret=interpret,
        grid=(NB,),
        in_specs=[
            pl.BlockSpec((Bn, 16), lambda i: (i, 0)),
            pl.BlockSpec((2, 1, Bn, 16), lambda i: (0, 0, i, 0)),
            pl.BlockSpec((2, 1, Bn, 16), lambda i: (0, 0, i, 0)),
            pl.BlockSpec((16, 112), lambda i: (0, 0)),
            pl.BlockSpec((1, 112), lambda i: (0, 0)),
        ],
        out_specs=[pl.BlockSpec((Bn, 16), lambda i: (i, 0)) for _ in range(7)],
        out_shape=[jax.ShapeDtypeStruct((NP, 16), jnp.float32) for _ in range(7)],
    )
    tc_final = pl.pallas_call(
        _tc_final_body,
        interpret=interpret,
        grid=(NB,),
        in_specs=(
            [pl.BlockSpec((Bn, 16), lambda i: (i, 0)) for _ in range(7)] + [
                pl.BlockSpec((2, 7, Bn, 16), lambda i: (0, 0, i, 0)),
                pl.BlockSpec((2, 1, Bn, 16), lambda i: (0, 0, i, 0)),
                pl.BlockSpec((1, 1, Bn), lambda i: (i, 0, 0)),
                pl.BlockSpec((112, 200), lambda i: (0, 0)),
                pl.BlockSpec((1, 200), lambda i: (0, 0)),
                pl.BlockSpec((200, 100), lambda i: (0, 0)),
                pl.BlockSpec((1, 100), lambda i: (0, 0)),
                pl.BlockSpec((100, 128), lambda i: (0, 0)),
                pl.BlockSpec((1, 128), lambda i: (0, 0)),
            ]
        ),
        out_specs=pl.BlockSpec((G, 128), lambda i: (0, 0)),
        out_shape=jax.ShapeDtypeStruct((G, 128), jnp.float32),
        scratch_shapes=[
            pltpu.VMEM((G, 200), jnp.float32),
            pltpu.VMEM((G, 128), jnp.float32),
        ],
    )
    return tc_prep, tc_mid, tc_final


def kernel(x, edge_index, batch, W1, b1, W2, b2, Wl1, bl1, Wl2, bl2):
    f32 = jnp.float32
    src = edge_index[0]
    dst = edge_index[1]
    epad = jnp.full((Ep - E,), N, jnp.int32)
    srcp = jnp.concatenate([src, epad]).reshape(R, 128)
    dstp = jnp.concatenate([dst, epad]).reshape(R, 128)
    xpad = jnp.zeros((NP, 16), f32).at[:N, :3].set(x.astype(f32))
    zeros16 = jnp.zeros((NP, 16), f32)
    onesv = jnp.ones((KB, 128, 16), f32)
    batchp = jnp.concatenate(
        [batch, jnp.full((NP - N,), G, jnp.int32)]).reshape(NB, 1, Bn)
    W1p = jnp.zeros((16, 112), f32).at[:3, :100].set(W1)
    b1p = jnp.zeros((1, 112), f32).at[0, :100].set(b1)
    W2p = jnp.zeros((112, 200), f32).at[:100, :].set(W2)
    b2r = b2.reshape(1, 200)
    bl1r = bl1.reshape(1, 100)
    Wl2p = jnp.zeros((100, 128), f32).at[:, 0:1].set(Wl2)
    bl2p = jnp.broadcast_to(bl2.reshape(1, 1), (1, 128))

    tc_prep, tc_mid, tc_final = _tc_kernels()
    degpart = _sc_pass(1, 16, False)(dstp, onesv, zeros16)
    xs = tc_prep(degpart, xpad)
    aggx = _sc_pass(1, 16, True)(xs, srcp, dstp, zeros16)
    ts = tc_mid(xs, aggx, degpart, W1p, b1p)
    aggh = _sc_pass(7, 16, True)(*ts, srcp, dstp, zeros16)
    out = tc_final(*ts, aggh, degpart, batchp, W2p, b2r, Wl1, bl1r,
                   Wl2p, bl2p)
    return out[:, 0]
